# Initial kernel scaffold; baseline (speedup 1.0000x reference)
import jax
import jax.numpy as jnp
from jax.experimental import pallas as pl


def kernel(x, edge_index, w1, b1, gamma, beta, wc, bc, wo, bo):
    raise NotImplementedError("write your pallas kernel here")


# speedup vs baseline: 12.6619x; 12.6619x over previous
import functools

import jax
import jax.numpy as jnp
from jax import lax
from jax.experimental import pallas as pl
from jax.experimental.pallas import tpu as pltpu
from jax.experimental.pallas import tpu_sc as plsc

N = 10000
E = 320000
D = 128
MU = 0.1
P_EXP = 2.0
CMU = 2.0 * MU / P_EXP
K_ITERS = 2
EPS = 1e-6
BN_EPS = 1e-5

NC = 2
NS = 16
NW = NC * NS
EPW = E // NW
B = 125
ITERS = EPW // B
RPT = 624

_mesh = plsc.VectorSubcoreMesh(core_axis_name="c", subcore_axis_name="s")


def _copy_rows(src, dst, s):
    pltpu.sync_copy(src.at[pl.ds(s * RPT, RPT)], dst.at[pl.ds(s * RPT, RPT)])

    @pl.when(s == NS - 1)
    def _():
        pltpu.sync_copy(src.at[pl.ds(NS * RPT, N - NS * RPT)],
                        dst.at[pl.ds(NS * RPT, N - NS * RPT)])


DW = D


@functools.partial(
    pl.kernel,
    mesh=_mesh,
    out_type=jax.ShapeDtypeStruct((NC, N, DW), jnp.float32),
    scratch_types=[
        pltpu.VMEM((ITERS, B), jnp.int32),
        pltpu.VMEM((B, DW), jnp.float32),
        pltpu.VMEM_SHARED((N, DW), jnp.float32),
    ],
)
def _deg_kernel(row_hbm, ones_hbm, zeros_hbm, out_hbm, row_v, ones_v, deg_sh):
    c = lax.axis_index("c")
    s = lax.axis_index("s")
    wid = s * NC + c

    _copy_rows(zeros_hbm, deg_sh, s)
    pltpu.sync_copy(ones_hbm, ones_v)
    pltpu.sync_copy(row_hbm.at[pl.ds(wid * ITERS, ITERS)], row_v)
    plsc.subcore_barrier()

    def body(j, _):
        pltpu.sync_copy(ones_v, deg_sh.at[row_v.at[j]], add=True)
        return 0

    lax.fori_loop(0, ITERS, body, 0)
    plsc.subcore_barrier()
    _copy_rows(deg_sh, out_hbm.at[c], s)


@functools.partial(
    pl.kernel,
    mesh=_mesh,
    out_type=jax.ShapeDtypeStruct((NC, N, D), jnp.float32),
    scratch_types=[
        pltpu.VMEM((ITERS, B), jnp.int32),
        pltpu.VMEM((ITERS, B), jnp.int32),
        pltpu.VMEM((B, D), jnp.float32),
        pltpu.VMEM_SHARED((N, D), jnp.float32),
        pltpu.SemaphoreType.DMA,
    ],
)
def _spmm_kernel(fn_hbm, col_hbm, row_hbm, zeros_hbm, out_hbm,
                 col_v, row_v, rows_v, agg_sh, gsem):
    c = lax.axis_index("c")
    s = lax.axis_index("s")
    wid = s * NC + c

    _copy_rows(zeros_hbm, agg_sh, s)
    pltpu.sync_copy(col_hbm.at[pl.ds(wid * ITERS, ITERS)], col_v)
    pltpu.sync_copy(row_hbm.at[pl.ds(wid * ITERS, ITERS)], row_v)
    plsc.subcore_barrier()

    def body(j, _):
        pltpu.async_copy(fn_hbm.at[col_v.at[j]], rows_v, gsem).wait()
        pltpu.sync_copy(rows_v, agg_sh.at[row_v.at[j]], add=True)
        return 0

    lax.fori_loop(0, ITERS, body, 0)
    plsc.subcore_barrier()
    _copy_rows(agg_sh, out_hbm.at[c], s)


RB = 2000
GRID = N // RB


def _lin1_body(x_ref, w_ref, sv_ref, bv_ref, o_ref):
    h = jnp.dot(x_ref[...], w_ref[...], preferred_element_type=jnp.float32)
    o_ref[...] = jnp.maximum(h * sv_ref[...] + bv_ref[...], 0.0)


def _prep_body(deg_ref, dinv_ref, ad_ref, bt_ref):
    v = deg_ref[...]
    deg_raw = v[0, :, 0:1] + v[1, :, 0:1]
    deg_c = jnp.maximum(deg_raw, EPS)
    dinv = lax.rsqrt(deg_c)
    alpha = 1.0 / (deg_raw / deg_c + CMU)
    dinv_ref[...] = dinv
    ad_ref[...] = alpha * dinv
    bt_ref[...] = CMU * alpha


def _convpre_body(h_ref, w_ref, bv_ref, dinv_ref, f0_ref, fn_ref):
    f0 = jnp.dot(h_ref[...], w_ref[...],
                 preferred_element_type=jnp.float32) + bv_ref[...]
    f0_ref[...] = f0
    fn_ref[...] = f0 * dinv_ref[...]


def _mid_body(a_ref, f0_ref, ad_ref, bt_ref, dinv_ref, fn_ref):
    f = ad_ref[...] * (a_ref[0] + a_ref[1]) + bt_ref[...] * f0_ref[...]
    fn_ref[...] = f * dinv_ref[...]


def _bnmm_body(a_ref, f0_ref, ad_ref, bt_ref, sv_ref, bb_ref, w_ref,
               bv_ref, dinv_ref, f0n_ref, fnn_ref):
    f = ad_ref[...] * (a_ref[0] + a_ref[1]) + bt_ref[...] * f0_ref[...]
    h = jnp.maximum(f * sv_ref[...] + bb_ref[...], 0.0)
    f0n = jnp.dot(h, w_ref[...],
                  preferred_element_type=jnp.float32) + bv_ref[...]
    f0n_ref[...] = f0n
    fnn_ref[...] = f0n * dinv_ref[...]


def _final_body(a_ref, f0_ref, ad_ref, bt_ref, o_ref):
    f = ad_ref[...] * (a_ref[0] + a_ref[1]) + bt_ref[...] * f0_ref[...]
    m = jnp.max(f, axis=1, keepdims=True)
    e = jnp.exp(f - m)
    lse = jnp.log(jnp.sum(e, axis=1, keepdims=True)) + m
    o_ref[...] = f - lse


_rowblk = pl.BlockSpec((RB, D), lambda i: (i, 0))
_aggblk = pl.BlockSpec((NC, RB, D), lambda i: (0, i, 0))
_wblk = pl.BlockSpec((D, D), lambda i: (0, 0))
_vecblk = pl.BlockSpec((1, D), lambda i: (0, 0))
_nodeblk = pl.BlockSpec((RB, 1), lambda i: (i, 0))
_nd = jax.ShapeDtypeStruct((N, D), jnp.float32)
_n1 = jax.ShapeDtypeStruct((N, 1), jnp.float32)


def _lin1(x, w, sv, bv):
    return pl.pallas_call(
        _lin1_body, grid=(GRID,),
        in_specs=[_rowblk, _wblk, _vecblk, _vecblk],
        out_specs=_rowblk, out_shape=_nd)(x, w, sv, bv)


def _prep(deg):
    return pl.pallas_call(
        _prep_body, grid=(GRID,),
        in_specs=[pl.BlockSpec((NC, RB, DW), lambda i: (0, i, 0))],
        out_specs=[_nodeblk, _nodeblk, _nodeblk],
        out_shape=[_n1, _n1, _n1])(deg)


def _convpre(h, w, bv, dinv):
    return pl.pallas_call(
        _convpre_body, grid=(GRID,),
        in_specs=[_rowblk, _wblk, _vecblk, _nodeblk],
        out_specs=[_rowblk, _rowblk],
        out_shape=[_nd, _nd])(h, w, bv, dinv)


def _mid(agg, f0, ad, bt, dinv):
    return pl.pallas_call(
        _mid_body, grid=(GRID,),
        in_specs=[_aggblk, _rowblk, _nodeblk, _nodeblk, _nodeblk],
        out_specs=_rowblk, out_shape=_nd)(agg, f0, ad, bt, dinv)


def _bnmm(agg, f0, ad, bt, sv, bb, w, bv, dinv):
    return pl.pallas_call(
        _bnmm_body, grid=(GRID,),
        in_specs=[_aggblk, _rowblk, _nodeblk, _nodeblk, _vecblk, _vecblk,
                  _wblk, _vecblk, _nodeblk],
        out_specs=[_rowblk, _rowblk],
        out_shape=[_nd, _nd])(agg, f0, ad, bt, sv, bb, w, bv, dinv)


def _final(agg, f0, ad, bt):
    return pl.pallas_call(
        _final_body, grid=(GRID,),
        in_specs=[_aggblk, _rowblk, _nodeblk, _nodeblk],
        out_specs=_rowblk, out_shape=_nd)(agg, f0, ad, bt)


def kernel(x, edge_index, w1, b1, gamma, beta, wc, bc, wo, bo):
    row = edge_index[0]
    col = edge_index[1]
    row_r = row.reshape(NW * ITERS, B)
    col_r = col.reshape(NW * ITERS, B)
    zeros2d = jnp.zeros((N, D), jnp.float32)

    deg = _deg_kernel(row_r, jnp.ones((B, DW), jnp.float32),
                      jnp.zeros((N, DW), jnp.float32))
    dinv, ad, bt = _prep(deg)

    s = gamma * (1.0 / jnp.sqrt(1.0 + BN_EPS))
    sv = s.reshape(1, D)
    bv1 = (b1 * s + beta).reshape(1, D)
    bb = beta.reshape(1, D)

    h = _lin1(x, w1, sv, bv1)

    f0, fn = _convpre(h, wc, bc.reshape(1, D), dinv)
    agg = _spmm_kernel(fn, col_r, row_r, zeros2d)
    fn = _mid(agg, f0, ad, bt, dinv)
    agg = _spmm_kernel(fn, col_r, row_r, zeros2d)
    f0, fn = _bnmm(agg, f0, ad, bt, sv, bb, wo, bo.reshape(1, D), dinv)

    agg = _spmm_kernel(fn, col_r, row_r, zeros2d)
    fn = _mid(agg, f0, ad, bt, dinv)
    agg = _spmm_kernel(fn, col_r, row_r, zeros2d)
    return _final(agg, f0, ad, bt)


# speedup vs baseline: 12.6752x; 1.0010x over previous
import functools

import jax
import jax.numpy as jnp
from jax import lax
from jax.experimental import pallas as pl
from jax.experimental.pallas import tpu as pltpu
from jax.experimental.pallas import tpu_sc as plsc

N = 10000
E = 320000
D = 128
MU = 0.1
P_EXP = 2.0
CMU = 2.0 * MU / P_EXP
K_ITERS = 2
EPS = 1e-6
BN_EPS = 1e-5

NC = 2
NS = 16
NW = NC * NS
EPW = E // NW
B = 125
ITERS = EPW // B
RPT = 624

_mesh = plsc.VectorSubcoreMesh(core_axis_name="c", subcore_axis_name="s")


def _copy_rows(src, dst, s):
    pltpu.sync_copy(src.at[pl.ds(s * RPT, RPT)], dst.at[pl.ds(s * RPT, RPT)])

    @pl.when(s == NS - 1)
    def _():
        pltpu.sync_copy(src.at[pl.ds(NS * RPT, N - NS * RPT)],
                        dst.at[pl.ds(NS * RPT, N - NS * RPT)])


DW = D


@functools.partial(
    pl.kernel,
    mesh=_mesh,
    out_type=jax.ShapeDtypeStruct((NC, N, DW), jnp.float32),
    scratch_types=[
        pltpu.VMEM((ITERS, B), jnp.int32),
        pltpu.VMEM((B, DW), jnp.float32),
        pltpu.VMEM_SHARED((N, DW), jnp.float32),
    ],
)
def _deg_kernel(row_hbm, ones_hbm, zeros_hbm, out_hbm, row_v, ones_v, deg_sh):
    c = lax.axis_index("c")
    s = lax.axis_index("s")
    wid = s * NC + c

    _copy_rows(zeros_hbm, deg_sh, s)
    pltpu.sync_copy(ones_hbm, ones_v)
    pltpu.sync_copy(row_hbm.at[wid], row_v)
    plsc.subcore_barrier()

    def body(j, _):
        pltpu.sync_copy(ones_v, deg_sh.at[row_v.at[j]], add=True)
        return 0

    lax.fori_loop(0, ITERS, body, 0)
    plsc.subcore_barrier()
    _copy_rows(deg_sh, out_hbm.at[c], s)


@functools.partial(
    pl.kernel,
    mesh=_mesh,
    out_type=jax.ShapeDtypeStruct((NC, N, D), jnp.float32),
    scratch_types=[
        pltpu.VMEM((ITERS, B), jnp.int32),
        pltpu.VMEM((ITERS, B), jnp.int32),
        pltpu.VMEM((B, D), jnp.float32),
        pltpu.VMEM_SHARED((N, D), jnp.float32),
        pltpu.SemaphoreType.DMA,
        pltpu.SemaphoreType.DMA,
    ],
)
def _spmm_kernel(fn_hbm, col_hbm, row_hbm, zeros_hbm, out_hbm,
                 col_v, row_v, ring, agg_sh, gsem, ssem):
    c = lax.axis_index("c")
    s = lax.axis_index("s")
    wid = s * NC + c

    _copy_rows(zeros_hbm, agg_sh, s)
    pltpu.sync_copy(col_hbm.at[wid], col_v)
    pltpu.sync_copy(row_hbm.at[wid], row_v)
    plsc.subcore_barrier()

    def body(j, _):
        pltpu.async_copy(fn_hbm.at[col_v.at[j]], ring, gsem).wait()
        pltpu.sync_copy(ring, agg_sh.at[row_v.at[j]], add=True)
        return 0

    lax.fori_loop(0, ITERS, body, 0)
    plsc.subcore_barrier()
    _copy_rows(agg_sh, out_hbm.at[c], s)


RB = 2000
GRID = N // RB


def _lin1_body(x_ref, w_ref, sv_ref, bv_ref, o_ref):
    h = jnp.dot(x_ref[...], w_ref[...], preferred_element_type=jnp.float32)
    o_ref[...] = jnp.maximum(h * sv_ref[...] + bv_ref[...], 0.0)


def _prep_body(deg_ref, dinv_ref, ad_ref, bt_ref):
    v = deg_ref[...]
    deg_raw = v[0, :, 0:1] + v[1, :, 0:1]
    deg_c = jnp.maximum(deg_raw, EPS)
    dinv = lax.rsqrt(deg_c)
    alpha = 1.0 / (deg_raw / deg_c + CMU)
    dinv_ref[...] = dinv
    ad_ref[...] = alpha * dinv
    bt_ref[...] = CMU * alpha


def _convpre_body(h_ref, w_ref, bv_ref, dinv_ref, f0_ref, fn_ref):
    f0 = jnp.dot(h_ref[...], w_ref[...],
                 preferred_element_type=jnp.float32) + bv_ref[...]
    f0_ref[...] = f0
    fn_ref[...] = f0 * dinv_ref[...]


def _mid_body(a_ref, f0_ref, ad_ref, bt_ref, dinv_ref, fn_ref):
    f = ad_ref[...] * (a_ref[0] + a_ref[1]) + bt_ref[...] * f0_ref[...]
    fn_ref[...] = f * dinv_ref[...]


def _bnmm_body(a_ref, f0_ref, ad_ref, bt_ref, sv_ref, bb_ref, w_ref,
               bv_ref, dinv_ref, f0n_ref, fnn_ref):
    f = ad_ref[...] * (a_ref[0] + a_ref[1]) + bt_ref[...] * f0_ref[...]
    h = jnp.maximum(f * sv_ref[...] + bb_ref[...], 0.0)
    f0n = jnp.dot(h, w_ref[...],
                  preferred_element_type=jnp.float32) + bv_ref[...]
    f0n_ref[...] = f0n
    fnn_ref[...] = f0n * dinv_ref[...]


def _final_body(a_ref, f0_ref, ad_ref, bt_ref, o_ref):
    f = ad_ref[...] * (a_ref[0] + a_ref[1]) + bt_ref[...] * f0_ref[...]
    m = jnp.max(f, axis=1, keepdims=True)
    e = jnp.exp(f - m)
    lse = jnp.log(jnp.sum(e, axis=1, keepdims=True)) + m
    o_ref[...] = f - lse


_rowblk = pl.BlockSpec((RB, D), lambda i: (i, 0))
_aggblk = pl.BlockSpec((NC, RB, D), lambda i: (0, i, 0))
_wblk = pl.BlockSpec((D, D), lambda i: (0, 0))
_vecblk = pl.BlockSpec((1, D), lambda i: (0, 0))
_nodeblk = pl.BlockSpec((RB, 1), lambda i: (i, 0))
_nd = jax.ShapeDtypeStruct((N, D), jnp.float32)
_n1 = jax.ShapeDtypeStruct((N, 1), jnp.float32)


def _lin1(x, w, sv, bv):
    return pl.pallas_call(
        _lin1_body, grid=(GRID,),
        in_specs=[_rowblk, _wblk, _vecblk, _vecblk],
        out_specs=_rowblk, out_shape=_nd)(x, w, sv, bv)


def _prep(deg):
    return pl.pallas_call(
        _prep_body, grid=(GRID,),
        in_specs=[pl.BlockSpec((NC, RB, DW), lambda i: (0, i, 0))],
        out_specs=[_nodeblk, _nodeblk, _nodeblk],
        out_shape=[_n1, _n1, _n1])(deg)


def _convpre(h, w, bv, dinv):
    return pl.pallas_call(
        _convpre_body, grid=(GRID,),
        in_specs=[_rowblk, _wblk, _vecblk, _nodeblk],
        out_specs=[_rowblk, _rowblk],
        out_shape=[_nd, _nd])(h, w, bv, dinv)


def _mid(agg, f0, ad, bt, dinv):
    return pl.pallas_call(
        _mid_body, grid=(GRID,),
        in_specs=[_aggblk, _rowblk, _nodeblk, _nodeblk, _nodeblk],
        out_specs=_rowblk, out_shape=_nd)(agg, f0, ad, bt, dinv)


def _bnmm(agg, f0, ad, bt, sv, bb, w, bv, dinv):
    return pl.pallas_call(
        _bnmm_body, grid=(GRID,),
        in_specs=[_aggblk, _rowblk, _nodeblk, _nodeblk, _vecblk, _vecblk,
                  _wblk, _vecblk, _nodeblk],
        out_specs=[_rowblk, _rowblk],
        out_shape=[_nd, _nd])(agg, f0, ad, bt, sv, bb, w, bv, dinv)


def _final(agg, f0, ad, bt):
    return pl.pallas_call(
        _final_body, grid=(GRID,),
        in_specs=[_aggblk, _rowblk, _nodeblk, _nodeblk],
        out_specs=_rowblk, out_shape=_nd)(agg, f0, ad, bt)


def kernel(x, edge_index, w1, b1, gamma, beta, wc, bc, wo, bo):
    row = edge_index[0]
    col = edge_index[1]
    row_r = row.reshape(NW, ITERS, B)
    col_r = col.reshape(NW, ITERS, B)
    zeros2d = jnp.zeros((N, D), jnp.float32)

    deg = _deg_kernel(row_r, jnp.ones((B, DW), jnp.float32),
                      jnp.zeros((N, DW), jnp.float32))
    dinv, ad, bt = _prep(deg)

    s = gamma * (1.0 / jnp.sqrt(1.0 + BN_EPS))
    sv = s.reshape(1, D)
    bv1 = (b1 * s + beta).reshape(1, D)
    bb = beta.reshape(1, D)

    h = _lin1(x, w1, sv, bv1)

    f0, fn = _convpre(h, wc, bc.reshape(1, D), dinv)
    agg = _spmm_kernel(fn, col_r, row_r, zeros2d)
    fn = _mid(agg, f0, ad, bt, dinv)
    agg = _spmm_kernel(fn, col_r, row_r, zeros2d)
    f0, fn = _bnmm(agg, f0, ad, bt, sv, bb, wo, bo.reshape(1, D), dinv)

    agg = _spmm_kernel(fn, col_r, row_r, zeros2d)
    fn = _mid(agg, f0, ad, bt, dinv)
    agg = _spmm_kernel(fn, col_r, row_r, zeros2d)
    return _final(agg, f0, ad, bt)


# speedup vs baseline: 12.7740x; 1.0078x over previous
import functools

import jax
import jax.numpy as jnp
from jax import lax
from jax.experimental import pallas as pl
from jax.experimental.pallas import tpu as pltpu
from jax.experimental.pallas import tpu_sc as plsc

N = 10000
E = 320000
D = 128
MU = 0.1
P_EXP = 2.0
CMU = 2.0 * MU / P_EXP
K_ITERS = 2
EPS = 1e-6
BN_EPS = 1e-5

NC = 2
NS = 16
NW = NC * NS
EPW = E // NW
B = 125
ITERS = EPW // B
RPT = 624

_mesh = plsc.VectorSubcoreMesh(core_axis_name="c", subcore_axis_name="s")


def _copy_rows(src, dst, s):
    pltpu.sync_copy(src.at[pl.ds(s * RPT, RPT)], dst.at[pl.ds(s * RPT, RPT)])

    @pl.when(s == NS - 1)
    def _():
        pltpu.sync_copy(src.at[pl.ds(NS * RPT, N - NS * RPT)],
                        dst.at[pl.ds(NS * RPT, N - NS * RPT)])


DW = D


@functools.partial(
    pl.kernel,
    mesh=_mesh,
    out_type=jax.ShapeDtypeStruct((NC, N, DW), jnp.float32),
    scratch_types=[
        pltpu.VMEM((ITERS, B), jnp.int32),
        pltpu.VMEM((B, DW), jnp.float32),
        pltpu.VMEM_SHARED((N, DW), jnp.float32),
    ],
)
def _deg_kernel(row_hbm, ones_hbm, zeros_hbm, out_hbm, row_v, ones_v, deg_sh):
    c = lax.axis_index("c")
    s = lax.axis_index("s")
    wid = s * NC + c

    _copy_rows(zeros_hbm, deg_sh, s)
    pltpu.sync_copy(ones_hbm, ones_v)
    pltpu.sync_copy(row_hbm.at[wid], row_v)
    plsc.subcore_barrier()

    def body(j, _):
        pltpu.sync_copy(ones_v, deg_sh.at[row_v.at[j]], add=True)
        return 0

    lax.fori_loop(0, ITERS, body, 0)
    plsc.subcore_barrier()
    _copy_rows(deg_sh, out_hbm.at[c], s)


@functools.partial(
    pl.kernel,
    mesh=_mesh,
    out_type=jax.ShapeDtypeStruct((NC, N, D), jnp.float32),
    scratch_types=[
        pltpu.VMEM((ITERS, B), jnp.int32),
        pltpu.VMEM((ITERS, B), jnp.int32),
        pltpu.VMEM((B, D), jnp.float32),
        pltpu.VMEM_SHARED((N, D), jnp.float32),
        pltpu.SemaphoreType.DMA,
        pltpu.SemaphoreType.DMA,
    ],
)
def _spmm_kernel(fn_hbm, col_hbm, row_hbm, zeros_hbm, out_hbm,
                 col_v, row_v, ring, agg_sh, gsem, ssem):
    c = lax.axis_index("c")
    s = lax.axis_index("s")
    wid = s * NC + c

    _copy_rows(zeros_hbm, agg_sh, s)
    pltpu.sync_copy(col_hbm.at[wid], col_v)
    pltpu.sync_copy(row_hbm.at[wid], row_v)
    plsc.subcore_barrier()

    def body(j, _):
        pltpu.async_copy(fn_hbm.at[col_v.at[j]], ring, gsem).wait()
        pltpu.sync_copy(ring, agg_sh.at[row_v.at[j]], add=True)
        return 0

    lax.fori_loop(0, ITERS, body, 0)
    plsc.subcore_barrier()
    _copy_rows(agg_sh, out_hbm.at[c], s)


RB = 2000
GRID = N // RB


def _front_body(deg_ref, x_ref, w1_ref, sv_ref, bv_ref, wc_ref, bc_ref,
                dinv_ref, ad_ref, bt_ref, f0_ref, fn_ref):
    v = deg_ref[...]
    deg_raw = v[0, :, 0:1] + v[1, :, 0:1]
    deg_c = jnp.maximum(deg_raw, EPS)
    dinv = lax.rsqrt(deg_c)
    alpha = 1.0 / (deg_raw / deg_c + CMU)
    dinv_ref[...] = dinv
    ad_ref[...] = alpha * dinv
    bt_ref[...] = CMU * alpha
    h = jnp.dot(x_ref[...], w1_ref[...], preferred_element_type=jnp.float32)
    h = jnp.maximum(h * sv_ref[...] + bv_ref[...], 0.0)
    f0 = jnp.dot(h, wc_ref[...],
                 preferred_element_type=jnp.float32) + bc_ref[...]
    f0_ref[...] = f0
    fn_ref[...] = f0 * dinv


def _mid_body(a_ref, f0_ref, ad_ref, bt_ref, dinv_ref, fn_ref):
    f = ad_ref[...] * (a_ref[0] + a_ref[1]) + bt_ref[...] * f0_ref[...]
    fn_ref[...] = f * dinv_ref[...]


def _bnmm_body(a_ref, f0_ref, ad_ref, bt_ref, sv_ref, bb_ref, w_ref,
               bv_ref, dinv_ref, f0n_ref, fnn_ref):
    f = ad_ref[...] * (a_ref[0] + a_ref[1]) + bt_ref[...] * f0_ref[...]
    h = jnp.maximum(f * sv_ref[...] + bb_ref[...], 0.0)
    f0n = jnp.dot(h, w_ref[...],
                  preferred_element_type=jnp.float32) + bv_ref[...]
    f0n_ref[...] = f0n
    fnn_ref[...] = f0n * dinv_ref[...]


def _final_body(a_ref, f0_ref, ad_ref, bt_ref, o_ref):
    f = ad_ref[...] * (a_ref[0] + a_ref[1]) + bt_ref[...] * f0_ref[...]
    m = jnp.max(f, axis=1, keepdims=True)
    e = jnp.exp(f - m)
    lse = jnp.log(jnp.sum(e, axis=1, keepdims=True)) + m
    o_ref[...] = f - lse


_rowblk = pl.BlockSpec((RB, D), lambda i: (i, 0))
_aggblk = pl.BlockSpec((NC, RB, D), lambda i: (0, i, 0))
_wblk = pl.BlockSpec((D, D), lambda i: (0, 0))
_vecblk = pl.BlockSpec((1, D), lambda i: (0, 0))
_nodeblk = pl.BlockSpec((RB, 1), lambda i: (i, 0))
_nd = jax.ShapeDtypeStruct((N, D), jnp.float32)
_n1 = jax.ShapeDtypeStruct((N, 1), jnp.float32)


def _front(deg, x, w1, sv, bv, wc, bc):
    return pl.pallas_call(
        _front_body, grid=(GRID,),
        in_specs=[pl.BlockSpec((NC, RB, DW), lambda i: (0, i, 0)),
                  _rowblk, _wblk, _vecblk, _vecblk, _wblk, _vecblk],
        out_specs=[_nodeblk, _nodeblk, _nodeblk, _rowblk, _rowblk],
        out_shape=[_n1, _n1, _n1, _nd, _nd])(deg, x, w1, sv, bv, wc, bc)


def _mid(agg, f0, ad, bt, dinv):
    return pl.pallas_call(
        _mid_body, grid=(GRID,),
        in_specs=[_aggblk, _rowblk, _nodeblk, _nodeblk, _nodeblk],
        out_specs=_rowblk, out_shape=_nd)(agg, f0, ad, bt, dinv)


def _bnmm(agg, f0, ad, bt, sv, bb, w, bv, dinv):
    return pl.pallas_call(
        _bnmm_body, grid=(GRID,),
        in_specs=[_aggblk, _rowblk, _nodeblk, _nodeblk, _vecblk, _vecblk,
                  _wblk, _vecblk, _nodeblk],
        out_specs=[_rowblk, _rowblk],
        out_shape=[_nd, _nd])(agg, f0, ad, bt, sv, bb, w, bv, dinv)


def _final(agg, f0, ad, bt):
    return pl.pallas_call(
        _final_body, grid=(GRID,),
        in_specs=[_aggblk, _rowblk, _nodeblk, _nodeblk],
        out_specs=_rowblk, out_shape=_nd)(agg, f0, ad, bt)


def kernel(x, edge_index, w1, b1, gamma, beta, wc, bc, wo, bo):
    row = edge_index[0]
    col = edge_index[1]
    row_r = row.reshape(NW, ITERS, B)
    col_r = col.reshape(NW, ITERS, B)
    zeros2d = jnp.zeros((N, D), jnp.float32)

    deg = _deg_kernel(row_r, jnp.ones((B, DW), jnp.float32),
                      jnp.zeros((N, DW), jnp.float32))

    s = gamma * (1.0 / jnp.sqrt(1.0 + BN_EPS))
    sv = s.reshape(1, D)
    bv1 = (b1 * s + beta).reshape(1, D)
    bb = beta.reshape(1, D)

    dinv, ad, bt, f0, fn = _front(deg, x, w1, sv, bv1, wc, bc.reshape(1, D))
    agg = _spmm_kernel(fn, col_r, row_r, zeros2d)
    fn = _mid(agg, f0, ad, bt, dinv)
    agg = _spmm_kernel(fn, col_r, row_r, zeros2d)
    f0, fn = _bnmm(agg, f0, ad, bt, sv, bb, wo, bo.reshape(1, D), dinv)

    agg = _spmm_kernel(fn, col_r, row_r, zeros2d)
    fn = _mid(agg, f0, ad, bt, dinv)
    agg = _spmm_kernel(fn, col_r, row_r, zeros2d)
    return _final(agg, f0, ad, bt)


# speedup vs baseline: 12.7882x; 1.0011x over previous
import functools

import jax
import jax.numpy as jnp
from jax import lax
from jax.experimental import pallas as pl
from jax.experimental.pallas import tpu as pltpu
from jax.experimental.pallas import tpu_sc as plsc

N = 10000
E = 320000
D = 128
MU = 0.1
P_EXP = 2.0
CMU = 2.0 * MU / P_EXP
K_ITERS = 2
EPS = 1e-6
BN_EPS = 1e-5

NC = 2
NS = 16
NW = NC * NS
EPW = E // NW
B = 125
ITERS = EPW // B
RPT = 624

_mesh = plsc.VectorSubcoreMesh(core_axis_name="c", subcore_axis_name="s")


def _copy_rows(src, dst, s):
    pltpu.sync_copy(src.at[pl.ds(s * RPT, RPT)], dst.at[pl.ds(s * RPT, RPT)])

    @pl.when(s == NS - 1)
    def _():
        pltpu.sync_copy(src.at[pl.ds(NS * RPT, N - NS * RPT)],
                        dst.at[pl.ds(NS * RPT, N - NS * RPT)])


DW = D


@functools.partial(
    pl.kernel,
    mesh=_mesh,
    out_type=jax.ShapeDtypeStruct((NC, N, DW), jnp.float32),
    scratch_types=[
        pltpu.VMEM((ITERS, B), jnp.int32),
        pltpu.VMEM((B, DW), jnp.float32),
        pltpu.VMEM_SHARED((N, DW), jnp.float32),
    ],
)
def _deg_kernel(row_hbm, ones_hbm, zeros_hbm, out_hbm, row_v, ones_v, deg_sh):
    c = lax.axis_index("c")
    s = lax.axis_index("s")
    wid = s * NC + c

    _copy_rows(zeros_hbm, deg_sh, s)
    pltpu.sync_copy(ones_hbm, ones_v)
    pltpu.sync_copy(row_hbm.at[wid], row_v)
    plsc.subcore_barrier()

    def body(j, _):
        pltpu.sync_copy(ones_v, deg_sh.at[row_v.at[j]], add=True)
        return 0

    lax.fori_loop(0, ITERS, body, 0)
    plsc.subcore_barrier()
    _copy_rows(deg_sh, out_hbm.at[c], s)


@functools.partial(
    pl.kernel,
    mesh=_mesh,
    out_type=jax.ShapeDtypeStruct((NC, N, D), jnp.float32),
    scratch_types=[
        pltpu.VMEM((ITERS, B), jnp.int32),
        pltpu.VMEM((ITERS, B), jnp.int32),
        pltpu.VMEM((B, D), jnp.float32),
        pltpu.VMEM_SHARED((N, D), jnp.float32),
        pltpu.SemaphoreType.DMA,
    ],
)
def _spmm_kernel(fn_hbm, col_hbm, row_hbm, zeros_hbm, out_hbm,
                 col_v, row_v, rows_v, agg_sh, gsem):
    c = lax.axis_index("c")
    s = lax.axis_index("s")
    wid = s * NC + c

    _copy_rows(zeros_hbm, agg_sh, s)
    pltpu.sync_copy(col_hbm.at[wid], col_v)
    pltpu.sync_copy(row_hbm.at[wid], row_v)
    plsc.subcore_barrier()

    def body(j, _):
        pltpu.async_copy(fn_hbm.at[col_v.at[j]], rows_v, gsem).wait()
        pltpu.sync_copy(rows_v, agg_sh.at[row_v.at[j]], add=True)
        return 0

    lax.fori_loop(0, ITERS, body, 0)
    plsc.subcore_barrier()
    _copy_rows(agg_sh, out_hbm.at[c], s)


RB = 2000
GRID = N // RB


def _front_body(deg_ref, x_ref, w1_ref, sv_ref, bv_ref, wc_ref, bc_ref,
                dinv_ref, ad_ref, bt_ref, f0_ref, fn_ref):
    v = deg_ref[...]
    deg_raw = v[0, :, 0:1] + v[1, :, 0:1]
    deg_c = jnp.maximum(deg_raw, EPS)
    dinv = lax.rsqrt(deg_c)
    alpha = 1.0 / (deg_raw / deg_c + CMU)
    dinv_ref[...] = dinv
    ad_ref[...] = alpha * dinv
    bt_ref[...] = CMU * alpha
    h = jnp.dot(x_ref[...], w1_ref[...], preferred_element_type=jnp.float32)
    h = jnp.maximum(h * sv_ref[...] + bv_ref[...], 0.0)
    f0 = jnp.dot(h, wc_ref[...],
                 preferred_element_type=jnp.float32) + bc_ref[...]
    f0_ref[...] = f0
    fn_ref[...] = f0 * dinv


def _mid_body(a_ref, f0_ref, ad_ref, bt_ref, dinv_ref, fn_ref):
    f = ad_ref[...] * (a_ref[0] + a_ref[1]) + bt_ref[...] * f0_ref[...]
    fn_ref[...] = f * dinv_ref[...]


def _bnmm_body(a_ref, f0_ref, ad_ref, bt_ref, sv_ref, bb_ref, w_ref,
               bv_ref, dinv_ref, f0n_ref, fnn_ref):
    f = ad_ref[...] * (a_ref[0] + a_ref[1]) + bt_ref[...] * f0_ref[...]
    h = jnp.maximum(f * sv_ref[...] + bb_ref[...], 0.0)
    f0n = jnp.dot(h, w_ref[...],
                  preferred_element_type=jnp.float32) + bv_ref[...]
    f0n_ref[...] = f0n
    fnn_ref[...] = f0n * dinv_ref[...]


def _final_body(a_ref, f0_ref, ad_ref, bt_ref, o_ref):
    f = ad_ref[...] * (a_ref[0] + a_ref[1]) + bt_ref[...] * f0_ref[...]
    m = jnp.max(f, axis=1, keepdims=True)
    e = jnp.exp(f - m)
    lse = jnp.log(jnp.sum(e, axis=1, keepdims=True)) + m
    o_ref[...] = f - lse


_rowblk = pl.BlockSpec((RB, D), lambda i: (i, 0))
_aggblk = pl.BlockSpec((NC, RB, D), lambda i: (0, i, 0))
_wblk = pl.BlockSpec((D, D), lambda i: (0, 0))
_vecblk = pl.BlockSpec((1, D), lambda i: (0, 0))
_nodeblk = pl.BlockSpec((RB, 1), lambda i: (i, 0))
_nd = jax.ShapeDtypeStruct((N, D), jnp.float32)
_n1 = jax.ShapeDtypeStruct((N, 1), jnp.float32)


def _front(deg, x, w1, sv, bv, wc, bc):
    return pl.pallas_call(
        _front_body, grid=(GRID,),
        in_specs=[pl.BlockSpec((NC, RB, DW), lambda i: (0, i, 0)),
                  _rowblk, _wblk, _vecblk, _vecblk, _wblk, _vecblk],
        out_specs=[_nodeblk, _nodeblk, _nodeblk, _rowblk, _rowblk],
        out_shape=[_n1, _n1, _n1, _nd, _nd])(deg, x, w1, sv, bv, wc, bc)


def _mid(agg, f0, ad, bt, dinv):
    return pl.pallas_call(
        _mid_body, grid=(GRID,),
        in_specs=[_aggblk, _rowblk, _nodeblk, _nodeblk, _nodeblk],
        out_specs=_rowblk, out_shape=_nd)(agg, f0, ad, bt, dinv)


def _bnmm(agg, f0, ad, bt, sv, bb, w, bv, dinv):
    return pl.pallas_call(
        _bnmm_body, grid=(GRID,),
        in_specs=[_aggblk, _rowblk, _nodeblk, _nodeblk, _vecblk, _vecblk,
                  _wblk, _vecblk, _nodeblk],
        out_specs=[_rowblk, _rowblk],
        out_shape=[_nd, _nd])(agg, f0, ad, bt, sv, bb, w, bv, dinv)


def _final(agg, f0, ad, bt):
    return pl.pallas_call(
        _final_body, grid=(GRID,),
        in_specs=[_aggblk, _rowblk, _nodeblk, _nodeblk],
        out_specs=_rowblk, out_shape=_nd)(agg, f0, ad, bt)


def kernel(x, edge_index, w1, b1, gamma, beta, wc, bc, wo, bo):
    row = edge_index[0]
    col = edge_index[1]
    row_r = row.reshape(NW, ITERS, B)
    col_r = col.reshape(NW, ITERS, B)
    zeros2d = jnp.zeros((N, D), jnp.float32)

    deg = _deg_kernel(row_r, jnp.ones((B, DW), jnp.float32),
                      jnp.zeros((N, DW), jnp.float32))

    s = gamma * (1.0 / jnp.sqrt(1.0 + BN_EPS))
    sv = s.reshape(1, D)
    bv1 = (b1 * s + beta).reshape(1, D)
    bb = beta.reshape(1, D)

    dinv, ad, bt, f0, fn = _front(deg, x, w1, sv, bv1, wc, bc.reshape(1, D))
    agg = _spmm_kernel(fn, col_r, row_r, zeros2d)
    fn = _mid(agg, f0, ad, bt, dinv)
    agg = _spmm_kernel(fn, col_r, row_r, zeros2d)
    f0, fn = _bnmm(agg, f0, ad, bt, sv, bb, wo, bo.reshape(1, D), dinv)

    agg = _spmm_kernel(fn, col_r, row_r, zeros2d)
    fn = _mid(agg, f0, ad, bt, dinv)
    agg = _spmm_kernel(fn, col_r, row_r, zeros2d)
    return _final(agg, f0, ad, bt)
